# baseline (speedup 1.0000x reference)
import functools

import jax
import jax.numpy as jnp
from jax import lax
from jax.experimental import pallas as pl
from jax.experimental.pallas import tpu as pltpu
from jax.experimental.pallas import tpu_sc as plsc

_B = 1024
_CTX = 4
_D = 64
_LANES = 16


def _sc_embed_sum(idx_flat, emb_table):
    info = plsc.get_sparse_core_info()
    nc, ns = info.num_cores, info.num_subcores
    nw = nc * ns
    bpw = _B // nw
    rows = bpw * _CTX
    mesh = plsc.VectorSubcoreMesh(core_axis_name="c", subcore_axis_name="s")

    @functools.partial(
        pl.kernel,
        mesh=mesh,
        compiler_params=pltpu.CompilerParams(use_tc_tiling_on_sc=False),
        out_type=jax.ShapeDtypeStruct((_B, _D), jnp.float32),
        scratch_types=[
            pltpu.VMEM((rows,), jnp.int32),
            pltpu.VMEM((rows, _D), jnp.float32),
            pltpu.VMEM((bpw, _D), jnp.float32),
            pltpu.SemaphoreType.DMA,
        ],
    )
    def k(idx_hbm, table_hbm, out_hbm, idx_v, rows_v, acc_v, sem):
        wid = lax.axis_index("s") * nc + lax.axis_index("c")
        base = wid * rows
        pltpu.sync_copy(idx_hbm.at[pl.ds(base, rows)], idx_v)
        pltpu.async_copy(table_hbm.at[idx_v], rows_v, sem).wait()
        for i in range(bpw):
            for j in range(_D // _LANES):
                s = pl.ds(j * _LANES, _LANES)
                acc_v[i, s] = (
                    rows_v[i * _CTX, s]
                    + rows_v[i * _CTX + 1, s]
                    + rows_v[i * _CTX + 2, s]
                    + rows_v[i * _CTX + 3, s]
                )
        pltpu.sync_copy(acc_v, out_hbm.at[pl.ds(wid * bpw, bpw)])

    return k(idx_flat, emb_table)


def _tc_project(embeds, Wt, b2d, rows=8, nbuf=4):
    v = Wt.shape[1]
    nb = _B // rows

    def body(e_ref, w_ref, b_ref, o_hbm, scr, sems):
        j = pl.program_id(0)
        slot = lax.rem(j, nbuf)

        def copy(s, jj):
            return pltpu.make_async_copy(
                scr.at[s], o_hbm.at[pl.ds(jj * rows, rows)], sems.at[s]
            )

        @pl.when(j >= nbuf)
        def _():
            copy(slot, j - nbuf).wait()

        scr[slot] = (
            lax.dot_general(
                e_ref[...],
                w_ref[...],
                dimension_numbers=(((1,), (0,)), ((), ())),
                preferred_element_type=jnp.float32,
            )
            + b_ref[...]
        )

        for s in range(nbuf):
            @pl.when(slot == s)
            def _():
                copy(slot, j).start(priority=s % 2)

        @pl.when(j == nb - 1)
        def _():
            for k in range(1, min(nbuf, nb) + 1):
                copy((nb - k) % nbuf, 0).wait()

    return pl.pallas_call(
        body,
        grid=(nb,),
        in_specs=[
            pl.BlockSpec((rows, _D), lambda j: (j, 0)),
            pl.BlockSpec((_D, v), lambda j: (0, 0)),
            pl.BlockSpec((1, v), lambda j: (0, 0)),
        ],
        out_specs=pl.BlockSpec(memory_space=pl.ANY),
        out_shape=jax.ShapeDtypeStruct((_B, v), jnp.float32),
        scratch_shapes=[
            pltpu.VMEM((nbuf, rows, v), jnp.float32),
            pltpu.SemaphoreType.DMA((nbuf,)),
        ],
    )(embeds, Wt, b2d)


def kernel(inputs, emb_table, W, b):
    idx_flat = inputs.T.reshape(-1).astype(jnp.int32)
    embeds = _sc_embed_sum(idx_flat, emb_table)
    Wt = W.T.astype(jnp.bfloat16)
    return _tc_project(embeds.astype(jnp.bfloat16), Wt, b.reshape(1, -1))


# speedup vs baseline: 1.1233x; 1.1233x over previous
import functools

import jax
import jax.numpy as jnp
from jax import lax
from jax.experimental import pallas as pl
from jax.experimental.pallas import tpu as pltpu
from jax.experimental.pallas import tpu_sc as plsc

_B = 1024
_CTX = 4
_D = 64
_LANES = 16


def _sc_embed_sum(idx_flat, emb_table):
    info = plsc.get_sparse_core_info()
    nc, ns = info.num_cores, info.num_subcores
    nw = nc * ns
    bpw = _B // nw
    rows = bpw * _CTX
    mesh = plsc.VectorSubcoreMesh(core_axis_name="c", subcore_axis_name="s")

    @functools.partial(
        pl.kernel,
        mesh=mesh,
        compiler_params=pltpu.CompilerParams(use_tc_tiling_on_sc=False),
        out_type=jax.ShapeDtypeStruct((_B, _D), jnp.float32),
        scratch_types=[
            pltpu.VMEM((rows,), jnp.int32),
            pltpu.VMEM((rows, _D), jnp.float32),
            pltpu.VMEM((bpw, _D), jnp.float32),
            pltpu.SemaphoreType.DMA,
        ],
    )
    def k(idx_hbm, table_hbm, out_hbm, idx_v, rows_v, acc_v, sem):
        wid = lax.axis_index("s") * nc + lax.axis_index("c")
        base = wid * rows
        pltpu.sync_copy(idx_hbm.at[pl.ds(base, rows)], idx_v)
        pltpu.async_copy(table_hbm.at[idx_v], rows_v, sem).wait()
        for i in range(bpw):
            for j in range(_D // _LANES):
                s = pl.ds(j * _LANES, _LANES)
                acc_v[i, s] = (
                    rows_v[i * _CTX, s]
                    + rows_v[i * _CTX + 1, s]
                    + rows_v[i * _CTX + 2, s]
                    + rows_v[i * _CTX + 3, s]
                )
        pltpu.sync_copy(acc_v, out_hbm.at[pl.ds(wid * bpw, bpw)])

    return k(idx_flat, emb_table)


def _tc_project(embeds, W, b2d, tile_v=4096):
    v = W.shape[0]

    def body(e_ref, w_ref, b_ref, o_ref):
        o_ref[...] = (
            lax.dot_general(
                e_ref[...],
                w_ref[...],
                dimension_numbers=(((1,), (1,)), ((), ())),
                preferred_element_type=jnp.float32,
            )
            + b_ref[...]
        )

    return pl.pallas_call(
        body,
        grid=(pl.cdiv(v, tile_v),),
        in_specs=[
            pl.BlockSpec((_B, _D), lambda j: (0, 0)),
            pl.BlockSpec((tile_v, _D), lambda j: (j, 0)),
            pl.BlockSpec((1, tile_v), lambda j: (0, j)),
        ],
        out_specs=pl.BlockSpec((_B, tile_v), lambda j: (0, j)),
        out_shape=jax.ShapeDtypeStruct((_B, v), jnp.float32),
    )(embeds, W, b2d)


def kernel(inputs, emb_table, W, b):
    idx_flat = inputs.T.reshape(-1).astype(jnp.int32)
    embeds = _sc_embed_sum(idx_flat, emb_table)
    return _tc_project(embeds, W, b.reshape(1, -1))
